# baseline (speedup 1.0000x reference)
import functools

import jax
import jax.numpy as jnp
from jax.experimental import pallas as pl
from jax.experimental.pallas import tpu as pltpu


def _mm_body(a_ref, b_ref, o_ref, acc_ref, *, nk, relu):
    k = pl.program_id(1)

    @pl.when(k == 0)
    def _init():
        acc_ref[...] = jnp.zeros_like(acc_ref)

    acc_ref[...] += jnp.dot(
        a_ref[...], b_ref[...], preferred_element_type=jnp.float32
    )

    @pl.when(k == nk - 1)
    def _epilogue():
        r = acc_ref[...]
        if relu:
            r = jnp.maximum(r, 0.0)
        o_ref[...] = r.astype(o_ref.dtype)


def _mm(a, b, *, relu=False, out_dtype=jnp.bfloat16, bm=512, bk=2048):
    m, k = a.shape
    _, n = b.shape
    bk = min(bk, k)
    bm = min(bm, m)
    nk = k // bk
    grid = (m // bm, nk)
    return pl.pallas_call(
        functools.partial(_mm_body, nk=nk, relu=relu),
        grid=grid,
        in_specs=[
            pl.BlockSpec((bm, bk), lambda i, j: (i, j)),
            pl.BlockSpec((bk, n), lambda i, j: (j, 0)),
        ],
        out_specs=pl.BlockSpec((bm, n), lambda i, j: (i, 0)),
        out_shape=jax.ShapeDtypeStruct((m, n), out_dtype),
        scratch_shapes=[pltpu.VMEM((bm, n), jnp.float32)],
        compiler_params=pltpu.CompilerParams(
            dimension_semantics=("parallel", "arbitrary"),
        ),
    )(a, b)


def kernel(x_0, x_1, neighborhood_0_to_0, neighborhood_1_to_2,
           W0_0, W0_1, W12_0, W12_1):
    bf = jnp.bfloat16
    a00 = neighborhood_0_to_0.astype(bf)
    b12 = neighborhood_1_to_2.astype(bf)

    h = _mm(x_0.astype(bf), W0_0.astype(bf))
    h = _mm(a00, h, relu=True)
    h = _mm(h, W0_1.astype(bf))
    x0_out = _mm(a00, h, relu=True, out_dtype=jnp.float32)

    g = _mm(x_1.astype(bf), W12_1.astype(bf))
    x2_out = _mm(b12, g, relu=True, out_dtype=jnp.float32)

    return (x0_out, x_1, x2_out)


# speedup vs baseline: 1.6090x; 1.6090x over previous
import functools

import jax
import jax.numpy as jnp
from jax.experimental import pallas as pl
from jax.experimental.pallas import tpu as pltpu


def _mm_body(a_ref, b_ref, o_ref, acc_ref, *, nk, relu):
    k = pl.program_id(1)

    @pl.when(k == 0)
    def _init():
        acc_ref[...] = jnp.zeros_like(acc_ref)

    acc_ref[...] += jnp.dot(
        a_ref[...].astype(jnp.bfloat16),
        b_ref[...].astype(jnp.bfloat16),
        preferred_element_type=jnp.float32,
    )

    @pl.when(k == nk - 1)
    def _epilogue():
        r = acc_ref[...]
        if relu:
            r = jnp.maximum(r, 0.0)
        o_ref[...] = r.astype(o_ref.dtype)


def _mm(a, b, *, relu=False, out_dtype=jnp.bfloat16, bm=512, bk=2048):
    m, k = a.shape
    _, n = b.shape
    bk = min(bk, k)
    bm = min(bm, m)
    nk = k // bk
    grid = (m // bm, nk)
    return pl.pallas_call(
        functools.partial(_mm_body, nk=nk, relu=relu),
        grid=grid,
        in_specs=[
            pl.BlockSpec((bm, bk), lambda i, j: (i, j)),
            pl.BlockSpec((bk, n), lambda i, j: (j, 0)),
        ],
        out_specs=pl.BlockSpec((bm, n), lambda i, j: (i, 0)),
        out_shape=jax.ShapeDtypeStruct((m, n), out_dtype),
        scratch_shapes=[pltpu.VMEM((bm, n), jnp.float32)],
        compiler_params=pltpu.CompilerParams(
            dimension_semantics=("parallel", "arbitrary"),
        ),
    )(a, b)


def kernel(x_0, x_1, neighborhood_0_to_0, neighborhood_1_to_2,
           W0_0, W0_1, W12_0, W12_1):
    a00 = neighborhood_0_to_0
    b12 = neighborhood_1_to_2

    h = _mm(x_0, W0_0)
    h = _mm(a00, h, relu=True)
    h = _mm(h, W0_1)
    x0_out = _mm(a00, h, relu=True, out_dtype=jnp.float32)

    g = _mm(x_1, W12_1)
    x2_out = _mm(b12, g, relu=True, out_dtype=jnp.float32)

    return (x0_out, x_1, x2_out)


# speedup vs baseline: 2.1115x; 1.3123x over previous
import functools

import jax
import jax.numpy as jnp
from jax.experimental import pallas as pl
from jax.experimental.pallas import tpu as pltpu


def _mm_small_body(a_ref, b_ref, o_ref, *, relu):
    r = jnp.dot(
        a_ref[...].astype(jnp.bfloat16),
        b_ref[...].astype(jnp.bfloat16),
        preferred_element_type=jnp.float32,
    )
    if relu:
        r = jnp.maximum(r, 0.0)
    o_ref[...] = r.astype(o_ref.dtype)


def _mm_resident(a, b, *, relu=False, out_dtype=jnp.bfloat16, bm=512, bk=None):
    m, k = a.shape
    _, n = b.shape
    bm = min(bm, m)
    if bk is None or bk >= k:
        grid = (m // bm,)
        return pl.pallas_call(
            functools.partial(_mm_small_body, relu=relu),
            grid=grid,
            in_specs=[
                pl.BlockSpec((bm, k), lambda i: (i, 0)),
                pl.BlockSpec((k, n), lambda i: (0, 0)),
            ],
            out_specs=pl.BlockSpec((bm, n), lambda i: (i, 0)),
            out_shape=jax.ShapeDtypeStruct((m, n), out_dtype),
            compiler_params=pltpu.CompilerParams(
                dimension_semantics=("parallel",),
            ),
        )(a, b)

    nk = k // bk

    def body(a_ref, b_ref, o_ref, acc_ref):
        kk = pl.program_id(1)

        @pl.when(kk == 0)
        def _init():
            acc_ref[...] = jnp.zeros_like(acc_ref)

        acc_ref[...] += jnp.dot(
            a_ref[...].astype(jnp.bfloat16),
            b_ref[...].astype(jnp.bfloat16),
            preferred_element_type=jnp.float32,
        )

        @pl.when(kk == nk - 1)
        def _epilogue():
            r = acc_ref[...]
            if relu:
                r = jnp.maximum(r, 0.0)
            o_ref[...] = r.astype(o_ref.dtype)

    return pl.pallas_call(
        body,
        grid=(m // bm, nk),
        in_specs=[
            pl.BlockSpec((bm, bk), lambda i, j: (i, j)),
            pl.BlockSpec((bk, n), lambda i, j: (j, 0)),
        ],
        out_specs=pl.BlockSpec((bm, n), lambda i, j: (i, 0)),
        out_shape=jax.ShapeDtypeStruct((m, n), out_dtype),
        scratch_shapes=[pltpu.VMEM((bm, n), jnp.float32)],
        compiler_params=pltpu.CompilerParams(
            dimension_semantics=("parallel", "arbitrary"),
        ),
    )(a, b)


def kernel(x_0, x_1, neighborhood_0_to_0, neighborhood_1_to_2,
           W0_0, W0_1, W12_0, W12_1):
    a00 = neighborhood_0_to_0
    b12 = neighborhood_1_to_2

    h = _mm_resident(x_0, W0_0, bm=1024)
    h = _mm_resident(a00, h, relu=True)
    h = _mm_resident(h, W0_1, bm=1024)
    x0_out = _mm_resident(a00, h, relu=True, out_dtype=jnp.float32)

    g = _mm_resident(x_1, W12_1, bm=2048)
    x2_out = _mm_resident(b12, g, relu=True, out_dtype=jnp.float32, bm=256)

    return (x0_out, x_1, x2_out)


# speedup vs baseline: 2.4310x; 1.1513x over previous
import functools

import jax
import jax.numpy as jnp
from jax.experimental import pallas as pl
from jax.experimental.pallas import tpu as pltpu

BF = jnp.bfloat16


def _x0_body(a_ref, x0_ref, w0_ref, w1_ref, o_ref, abf_ref, xw_ref, h1_ref,
             *, bm):
    l = pl.program_id(0)
    i = pl.program_id(1)
    rows = pl.ds(i * bm, bm)

    @pl.when((l == 0) & (i == 0))
    def _xw0():
        xw_ref[...] = jnp.dot(
            x0_ref[...].astype(BF), w0_ref[...].astype(BF),
            preferred_element_type=jnp.float32).astype(BF)

    @pl.when(l == 0)
    def _layer0():
        ab = a_ref[...].astype(BF)
        abf_ref[rows, :] = ab
        h = jnp.dot(ab, xw_ref[...], preferred_element_type=jnp.float32)
        h1_ref[rows, :] = jnp.maximum(h, 0.0).astype(BF)

    @pl.when((l == 1) & (i == 0))
    def _xw1():
        xw_ref[...] = jnp.dot(
            h1_ref[...], w1_ref[...].astype(BF),
            preferred_element_type=jnp.float32).astype(BF)

    @pl.when(l == 1)
    def _layer1():
        r = jnp.dot(abf_ref[rows, :], xw_ref[...],
                    preferred_element_type=jnp.float32)
        o_ref[...] = jnp.maximum(r, 0.0)


def _x0_path(a00, x_0, w0, w1, *, bm=256):
    m, k = a00.shape
    _, d = x_0.shape
    ni = m // bm
    last = ni - 1
    return pl.pallas_call(
        functools.partial(_x0_body, bm=bm),
        grid=(2, ni),
        in_specs=[
            pl.BlockSpec((bm, k), lambda l, i: (jnp.where(l == 0, i, last), 0)),
            pl.BlockSpec((m, d), lambda l, i: (0, 0)),
            pl.BlockSpec((d, d), lambda l, i: (0, 0)),
            pl.BlockSpec((d, d), lambda l, i: (0, 0)),
        ],
        out_specs=pl.BlockSpec((bm, d), lambda l, i: (i, 0)),
        out_shape=jax.ShapeDtypeStruct((m, d), jnp.float32),
        scratch_shapes=[
            pltpu.VMEM((m, k), BF),
            pltpu.VMEM((m, d), BF),
            pltpu.VMEM((m, d), BF),
        ],
        compiler_params=pltpu.CompilerParams(
            dimension_semantics=("arbitrary", "arbitrary"),
        ),
    )(a00, x_0, w0, w1)


def _x2_body(b_ref, x1_ref, w_ref, o_ref, g_ref):
    i = pl.program_id(0)

    @pl.when(i == 0)
    def _g():
        g_ref[...] = jnp.dot(
            x1_ref[...].astype(BF), w_ref[...].astype(BF),
            preferred_element_type=jnp.float32).astype(BF)

    r = jnp.dot(b_ref[...].astype(BF), g_ref[...],
                preferred_element_type=jnp.float32)
    o_ref[...] = jnp.maximum(r, 0.0)


def _x2_path(b12, x_1, w, *, bm=256):
    m, k = b12.shape
    _, d = w.shape
    return pl.pallas_call(
        _x2_body,
        grid=(m // bm,),
        in_specs=[
            pl.BlockSpec((bm, k), lambda i: (i, 0)),
            pl.BlockSpec((k, x_1.shape[1]), lambda i: (0, 0)),
            pl.BlockSpec((x_1.shape[1], d), lambda i: (0, 0)),
        ],
        out_specs=pl.BlockSpec((bm, d), lambda i: (i, 0)),
        out_shape=jax.ShapeDtypeStruct((m, d), jnp.float32),
        scratch_shapes=[pltpu.VMEM((k, d), BF)],
        compiler_params=pltpu.CompilerParams(
            dimension_semantics=("arbitrary",),
        ),
    )(b12, x_1, w)


def kernel(x_0, x_1, neighborhood_0_to_0, neighborhood_1_to_2,
           W0_0, W0_1, W12_0, W12_1):
    x0_out = _x0_path(neighborhood_0_to_0, x_0, W0_0, W0_1)
    x2_out = _x2_path(neighborhood_1_to_2, x_1, W12_1)
    return (x0_out, x_1, x2_out)
